# baseline (speedup 1.0000x reference)
import functools

import jax
import jax.numpy as jnp
from jax.experimental import pallas as pl
from jax.experimental.pallas import tpu as pltpu

_KK = 31
_R = 256
_CT = 512
_P = 4
_G = 2


def _emb_kernel(f_ref, w0_ref, w1_ref, o_ref):
    h = jnp.maximum(f_ref[...] * w0_ref[...], 0.0) * w1_ref[...]
    nrm = jnp.sqrt(jnp.sum(h * h, axis=1, keepdims=True))
    o_ref[...] = h / jnp.maximum(nrm, 1e-12)


def _strip_kernel(ac_ref, ap_ref, bT_ref, o_ref, cand_ref, *, nt, n):
    i = pl.program_id(0)
    par = jax.lax.rem(i, 2)
    prev = 1 - par
    nc = _P * _G

    s8 = [cand_ref[prev, :, k * 128:(k + 1) * 128] for k in range(nc)]
    t = None
    for _ in range(_KK):
        t = jnp.max(s8[0], axis=1, keepdims=True)
        shift = s8[0] == t
        for k in range(nc - 1):
            s8[k] = jnp.where(shift, s8[k + 1], s8[k])
        s8[nc - 1] = jnp.where(shift, -1.0, s8[nc - 1])

    ap = ap_ref[...]
    for j in range(nt):
        start = j * _CT
        if start >= n:
            break
        w = min(_CT, n - start)
        tile = jnp.maximum(
            jnp.dot(ap, bT_ref[:, pl.ds(start, _CT)],
                    preferred_element_type=jnp.float32), 0.0)
        masked = jnp.where(tile >= t, tile, 0.0)
        o_ref[:, pl.ds(start, w)] = masked[:, :w]

    a = ac_ref[...]
    r = a.shape[0]
    slots = [[jnp.full((r, 128), -1.0, jnp.float32) for _ in range(_P)]
             for _ in range(_G)]
    nq = _CT // 128
    for j in range(nt):
        tile = jnp.maximum(
            jnp.dot(a, bT_ref[:, pl.ds(j * _CT, _CT)],
                    preferred_element_type=jnp.float32), 0.0)
        for q in range(nq):
            g = q * _G // nq
            cur = tile[:, q * 128:(q + 1) * 128]
            for s in range(_P):
                hi = jnp.maximum(slots[g][s], cur)
                cur = jnp.minimum(slots[g][s], cur)
                slots[g][s] = hi

    sn = slots[0] + slots[1]

    for ia, ja in [(0, 1), (2, 3), (4, 5), (6, 7),
                   (0, 2), (1, 3), (4, 6), (5, 7),
                   (1, 2), (5, 6),
                   (0, 4), (1, 5), (2, 6), (3, 7),
                   (2, 4), (3, 5),
                   (1, 2), (3, 4), (5, 6)]:
        hi = jnp.maximum(sn[ia], sn[ja])
        lo = jnp.minimum(sn[ia], sn[ja])
        sn[ia], sn[ja] = hi, lo

    for k in range(nc):
        cand_ref[par, :, k * 128:(k + 1) * 128] = sn[k]


def kernel(features, w0, w1):
    n, d = features.shape
    npad = -(-n // _CT) * _CT
    nt = npad // _CT
    nstrips = npad // _R

    f = features
    if npad != n:
        f = jnp.pad(features, ((0, npad - n), (0, 0)))

    rb = 1024 if npad % 1024 == 0 else _R
    emb = pl.pallas_call(
        _emb_kernel,
        grid=(npad // rb,),
        in_specs=[
            pl.BlockSpec((rb, d), lambda i: (i, 0)),
            pl.BlockSpec((1, d), lambda i: (0, 0)),
            pl.BlockSpec((1, d), lambda i: (0, 0)),
        ],
        out_specs=pl.BlockSpec((rb, d), lambda i: (i, 0)),
        out_shape=jax.ShapeDtypeStruct((npad, d), jnp.float32),
    )(f, w0.reshape(1, d), w1.reshape(1, d))

    embT = emb.T

    out = pl.pallas_call(
        functools.partial(_strip_kernel, nt=nt, n=n),
        grid=(nstrips + 1,),
        in_specs=[
            pl.BlockSpec(
                (_R, d), lambda i: (jnp.minimum(i, nstrips - 1), 0)),
            pl.BlockSpec((_R, d), lambda i: (jnp.maximum(i - 1, 0), 0)),
            pl.BlockSpec((d, npad), lambda i: (0, 0)),
        ],
        out_specs=pl.BlockSpec(
            (_R, n), lambda i: (jnp.maximum(i - 1, 0), 0)),
        out_shape=jax.ShapeDtypeStruct((n, n), jnp.float32),
        scratch_shapes=[
            pltpu.VMEM((2, _R, 128 * _P * _G), jnp.float32),
        ],
        compiler_params=pltpu.CompilerParams(
            dimension_semantics=("arbitrary",)),
    )(emb, emb, embT)

    return out


# speedup vs baseline: 1.2084x; 1.2084x over previous
import jax
import jax.numpy as jnp
from jax.experimental import pallas as pl
from jax.experimental.pallas import tpu as pltpu

_KK = 31
_R = 256
_CT = 512


def _emb_kernel(f_ref, w0_ref, w1_ref, o_ref):
    h = jnp.maximum(f_ref[...] * w0_ref[...], 0.0) * w1_ref[...]
    nrm = jnp.sqrt(jnp.sum(h * h, axis=1, keepdims=True))
    o_ref[...] = h / jnp.maximum(nrm, 1e-12)


_P = 4
_G = 2


def _ce(arrs, pairs):
    for x, y in pairs:
        hi = jnp.maximum(arrs[x], arrs[y])
        lo = jnp.minimum(arrs[x], arrs[y])
        arrs[x], arrs[y] = hi, lo


def _merge4(s, b4):
    u = [jnp.maximum(s[i], b4[3 - i]) for i in range(4)]
    _ce(u, [(0, 2), (1, 3), (0, 1), (2, 3)])
    return u


def _strip_kernel(a_ref, bT_ref, o_ref, strip_ref, *, nt, n):
    a = a_ref[...]
    r = a.shape[0]
    slots = [[jnp.full((r, 128), -1.0, jnp.float32) for _ in range(_P)]
             for _ in range(_G)]
    neg = jnp.full((r, 128), -1.0, jnp.float32)
    for jj in range(0, nt, 2):
        tiles = []
        for j in range(jj, min(jj + 2, nt)):
            tile = jnp.maximum(
                jnp.dot(a, bT_ref[:, pl.ds(j * _CT, _CT)],
                        preferred_element_type=jnp.float32), 0.0)
            strip_ref[:, pl.ds(j * _CT, _CT)] = tile
            tiles.append(tile)
        for g in range(_G):
            batch = [t[:, (2 * g + qq) * 128:(2 * g + qq + 1) * 128]
                     for t in tiles for qq in range(2)]
            if len(batch) == 4:
                _ce(batch, [(0, 1), (2, 3), (0, 2), (1, 3), (1, 2)])
            else:
                _ce(batch, [(0, 1)])
                batch = [batch[0], batch[1], neg, neg]
            slots[g] = _merge4(slots[g], batch)

    s8 = slots[0] + slots[1][::-1]
    _ce(s8, [(0, 4), (1, 5), (2, 6), (3, 7),
             (0, 2), (1, 3), (4, 6), (5, 7),
             (0, 1), (2, 3), (4, 5), (6, 7)])

    t = None
    for _ in range(_KK):
        t = jnp.max(s8[0], axis=1, keepdims=True)
        shift = s8[0] == t
        for k in range(7):
            s8[k] = jnp.where(shift, s8[k + 1], s8[k])
        s8[7] = jnp.where(shift, -1.0, s8[7])

    for j in range(nt):
        start = j * _CT
        if start >= n:
            break
        w = min(_CT, n - start)
        tile = strip_ref[:, pl.ds(start, _CT)]
        masked = jnp.where(tile >= t, tile, 0.0)
        o_ref[:, pl.ds(start, w)] = masked[:, :w]


def kernel(features, w0, w1):
    n, d = features.shape
    npad = -(-n // _CT) * _CT
    nt = npad // _CT

    f = features
    if npad != n:
        f = jnp.pad(features, ((0, npad - n), (0, 0)))

    rb = 1024 if npad % 1024 == 0 else _R
    emb = pl.pallas_call(
        _emb_kernel,
        grid=(npad // rb,),
        in_specs=[
            pl.BlockSpec((rb, d), lambda i: (i, 0)),
            pl.BlockSpec((1, d), lambda i: (0, 0)),
            pl.BlockSpec((1, d), lambda i: (0, 0)),
        ],
        out_specs=pl.BlockSpec((rb, d), lambda i: (i, 0)),
        out_shape=jax.ShapeDtypeStruct((npad, d), jnp.float32),
    )(f, w0.reshape(1, d), w1.reshape(1, d))

    embT = emb.T

    import functools
    out = pl.pallas_call(
        functools.partial(_strip_kernel, nt=nt, n=n),
        grid=(npad // _R,),
        in_specs=[
            pl.BlockSpec((_R, d), lambda i: (i, 0)),
            pl.BlockSpec((d, npad), lambda i: (0, 0)),
        ],
        out_specs=pl.BlockSpec((_R, n), lambda i: (i, 0)),
        out_shape=jax.ShapeDtypeStruct((n, n), jnp.float32),
        scratch_shapes=[pltpu.VMEM((_R, npad), jnp.float32)],
        compiler_params=pltpu.CompilerParams(
            dimension_semantics=("parallel",)),
    )(emb, embT)

    return out
